# Initial kernel scaffold; baseline (speedup 1.0000x reference)
import jax
import jax.numpy as jnp
from jax.experimental import pallas as pl


def kernel(x, edge_index, W1, b1, W2, b2):
    raise NotImplementedError("write your pallas kernel here")


# speedup vs baseline: 9.4565x; 9.4565x over previous
import functools

import jax
import jax.numpy as jnp
from jax import lax
from jax.experimental import pallas as pl
from jax.experimental.pallas import tpu as pltpu
from jax.experimental.pallas import tpu_sc as plsc

N = 10000
D = 128
DH = 256
E = 320000
NP = 10240
K = 128
NC, NS = 2, 16
NW = NC * NS
EP = 327680
NCH = EP // (NW * K)
RP = NP // NS
DW = 16

_mesh = plsc.VectorSubcoreMesh(
    core_axis_name="c", subcore_axis_name="s", num_cores=NC, num_subcores=NS)


@functools.partial(
    pl.kernel,
    out_type=jax.ShapeDtypeStruct((NC, NS, NP), jnp.float32),
    mesh=_mesh,
    scratch_types=[
        pltpu.VMEM((NCH, K), jnp.int32),
        pltpu.VMEM((NP,), jnp.float32),
    ],
    compiler_params=pltpu.CompilerParams(needs_layout_passes=False),
)
def _deg_kernel(dst_hbm, zeros_hbm, out_hbm, idx_v, hist_v):
    c = lax.axis_index("c")
    s = lax.axis_index("s")
    w = c * NS + s
    pltpu.sync_copy(dst_hbm.at[pl.ds(w * NCH, NCH)], idx_v)
    pltpu.sync_copy(zeros_hbm, hist_v)
    ones_v = jnp.ones((16,), jnp.float32)

    def body(ch, carry):
        def inner(j, carry2):
            idx = idx_v[ch, pl.ds(j * 16, 16)]
            plsc.addupdate_scatter(hist_v, [idx], ones_v)
            return carry2

        return lax.fori_loop(0, K // 16, inner, carry)

    lax.fori_loop(0, NCH, body, 0)
    pltpu.sync_copy(hist_v, out_hbm.at[c, s])


@functools.partial(
    pl.kernel,
    out_type=jax.ShapeDtypeStruct((NC, NP, D), jnp.float32),
    mesh=_mesh,
    scratch_types=[
        pltpu.VMEM((NCH, K), jnp.int32),
        pltpu.VMEM((NCH, K), jnp.int32),
        pltpu.VMEM((K, D), jnp.float32),
        pltpu.VMEM_SHARED((NP, D), jnp.float32),
        pltpu.SemaphoreType.DMA,
    ],
)
def _agg_kernel(tab_hbm, src_hbm, dst_hbm, zeros_hbm, out_hbm,
                src_v, dst_v, rows_v, acc_sh, sem):
    c = lax.axis_index("c")
    s = lax.axis_index("s")
    w = c * NS + s
    pltpu.sync_copy(src_hbm.at[pl.ds(w * NCH, NCH)], src_v)
    pltpu.sync_copy(dst_hbm.at[pl.ds(w * NCH, NCH)], dst_v)
    pltpu.sync_copy(zeros_hbm, acc_sh.at[pl.ds(s * RP, RP)])
    plsc.subcore_barrier()

    def body(ch, carry):
        pltpu.async_copy(tab_hbm.at[src_v.at[ch]], rows_v, sem).wait()
        pltpu.sync_copy(rows_v, acc_sh.at[dst_v.at[ch]], add=True)
        return carry

    lax.fori_loop(0, NCH, body, 0)
    plsc.subcore_barrier()
    pltpu.sync_copy(acc_sh.at[pl.ds(s * RP, RP)],
                    out_hbm.at[c, pl.ds(s * RP, RP)])


def _prep_body(degp_ref, x_ref, x1_ref, dis_ref):
    degp = degp_ref[...]
    deg = jnp.sum(degp.reshape(NC * NS, NP), axis=0) + 1.0
    dis = lax.rsqrt(deg)
    dis2 = jnp.broadcast_to(dis[:, None], (NP, D))
    dis_ref[...] = dis2
    x1_ref[0:N, :] = x_ref[...] * dis2[0:N, :]
    x1_ref[N:NP, :] = jnp.zeros((NP - N, D), jnp.float32)


_prep = pl.pallas_call(
    _prep_body,
    out_shape=(jax.ShapeDtypeStruct((NP, D), jnp.float32),
               jax.ShapeDtypeStruct((NP, D), jnp.float32)),
)

RB = 512
GB = NP // RB


def _mid_body(acc_ref, x1_ref, dis_ref, W1_ref, b1_ref, W2_ref, x2_ref):
    i = pl.program_id(0)
    dis = dis_ref[...]
    u1 = dis * (acc_ref[0] + acc_ref[1] + x1_ref[...])
    h1 = jnp.dot(u1, W1_ref[...], preferred_element_type=jnp.float32,
                 precision=lax.Precision.HIGHEST) + b1_ref[...]
    h1 = jnp.maximum(h1, 0.0)
    g2 = jnp.dot(dis[:, 0:1] * h1, W2_ref[...],
                 preferred_element_type=jnp.float32,
                 precision=lax.Precision.HIGHEST)
    rows = lax.broadcasted_iota(jnp.int32, (RB, D), 0) + i * RB
    x2_ref[...] = jnp.where(rows < N, g2, 0.0)


_mid = pl.pallas_call(
    _mid_body,
    grid=(GB,),
    in_specs=[
        pl.BlockSpec((NC, RB, D), lambda i: (0, i, 0)),
        pl.BlockSpec((RB, D), lambda i: (i, 0)),
        pl.BlockSpec((RB, D), lambda i: (i, 0)),
        pl.BlockSpec((D, DH), lambda i: (0, 0)),
        pl.BlockSpec((1, DH), lambda i: (0, 0)),
        pl.BlockSpec((DH, D), lambda i: (0, 0)),
    ],
    out_specs=pl.BlockSpec((RB, D), lambda i: (i, 0)),
    out_shape=jax.ShapeDtypeStruct((NP, D), jnp.float32),
)


def _fin_body(acc_ref, x2_ref, dis_ref, b2_ref, out_ref):
    u = dis_ref[...] * (acc_ref[0] + acc_ref[1] + x2_ref[...])
    out_ref[...] = jnp.maximum(u + b2_ref[...], 0.0)


_fin = pl.pallas_call(
    _fin_body,
    grid=(GB,),
    in_specs=[
        pl.BlockSpec((NC, RB, D), lambda i: (0, i, 0)),
        pl.BlockSpec((RB, D), lambda i: (i, 0)),
        pl.BlockSpec((RB, D), lambda i: (i, 0)),
        pl.BlockSpec((1, D), lambda i: (0, 0)),
    ],
    out_specs=pl.BlockSpec((RB, D), lambda i: (i, 0)),
    out_shape=jax.ShapeDtypeStruct((NP, D), jnp.float32),
)


def kernel(x, edge_index, W1, b1, W2, b2):
    pad = jnp.full((EP - E,), N, jnp.int32)
    src2 = jnp.concatenate([edge_index[0], pad]).reshape(EP // K, K)
    dst2 = jnp.concatenate([edge_index[1], pad]).reshape(EP // K, K)
    zerosN = jnp.zeros((NP,), jnp.float32)
    zerosD = jnp.zeros((RP, D), jnp.float32)

    degp = _deg_kernel(dst2, zerosN)
    x1, dis = _prep(degp, x)
    acc1 = _agg_kernel(x1, src2, dst2, zerosD)
    x2 = _mid(acc1, x1, dis, W1, b1.reshape(1, DH), W2)
    acc2 = _agg_kernel(x2, src2, dst2, zerosD)
    out = _fin(acc2, x2, dis, b2.reshape(1, D))
    return out[:N]


# speedup vs baseline: 10.4037x; 1.1002x over previous
import functools

import jax
import jax.numpy as jnp
from jax import lax
from jax.experimental import pallas as pl
from jax.experimental.pallas import tpu as pltpu
from jax.experimental.pallas import tpu_sc as plsc

N = 10000
D = 128
DH = 256
E = 320000
NP = 10240
K = 128
NC, NS = 2, 16
NW = NC * NS
EP = 327680
NCH = EP // (NW * K)
RP = NP // NS
DW = 16

_mesh = plsc.VectorSubcoreMesh(
    core_axis_name="c", subcore_axis_name="s", num_cores=NC, num_subcores=NS)


@functools.partial(
    pl.kernel,
    out_type=jax.ShapeDtypeStruct((NC, NS, NP), jnp.float32),
    mesh=_mesh,
    scratch_types=[
        pltpu.VMEM((NCH, K), jnp.int32),
        pltpu.VMEM((NP,), jnp.float32),
    ],
    compiler_params=pltpu.CompilerParams(needs_layout_passes=False),
)
def _deg_kernel(dst_hbm, zeros_hbm, out_hbm, idx_v, hist_v):
    c = lax.axis_index("c")
    s = lax.axis_index("s")
    w = c * NS + s
    pltpu.sync_copy(dst_hbm.at[pl.ds(w * NCH, NCH)], idx_v)
    pltpu.sync_copy(zeros_hbm, hist_v)
    ones_v = jnp.ones((16,), jnp.float32)

    def body(ch, carry):
        def inner(j, carry2):
            idx = idx_v[ch, pl.ds(j * 16, 16)]
            plsc.addupdate_scatter(hist_v, [idx], ones_v)
            return carry2

        return lax.fori_loop(0, K // 16, inner, carry)

    lax.fori_loop(0, NCH, body, 0)
    pltpu.sync_copy(hist_v, out_hbm.at[c, s])


@functools.partial(
    pl.kernel,
    out_type=jax.ShapeDtypeStruct((NC, NP, D), jnp.float32),
    mesh=_mesh,
    scratch_types=[
        pltpu.VMEM((NCH // 2, K), jnp.int32),
        pltpu.VMEM((NCH // 2, K), jnp.int32),
        pltpu.VMEM((K, D), jnp.float32),
        pltpu.VMEM((K, D), jnp.float32),
        pltpu.VMEM_SHARED((NP, D), jnp.float32),
        pltpu.SemaphoreType.DMA,
        pltpu.SemaphoreType.DMA,
    ],
)
def _agg_kernel(tab_hbm, src_hbm, dst_hbm, zeros_hbm, out_hbm,
                src_v, dst_v, rows0_v, rows1_v, acc_sh, sem0, sem1):
    c = lax.axis_index("c")
    s = lax.axis_index("s")
    w = c * NS + s
    NC2 = NCH // 2
    pltpu.sync_copy(zeros_hbm, acc_sh.at[pl.ds(s * RP, RP)])
    plsc.subcore_barrier()

    for p in range(2):
        pltpu.sync_copy(src_hbm.at[pl.ds(w * NCH + p * NC2, NC2)], src_v)
        pltpu.sync_copy(dst_hbm.at[pl.ds(w * NCH + p * NC2, NC2)], dst_v)
        pltpu.async_copy(tab_hbm.at[src_v.at[0]], rows0_v, sem0)

        def body(i, carry):
            ch0 = 2 * i
            ch1 = 2 * i + 1
            pltpu.async_copy(tab_hbm.at[src_v.at[ch1]], rows1_v, sem1)
            pltpu.make_async_copy(
                tab_hbm.at[src_v.at[ch0]], rows0_v, sem0).wait()
            pltpu.sync_copy(rows0_v, acc_sh.at[dst_v.at[ch0]], add=True)

            @pl.when(ch1 + 1 < NC2)
            def _():
                pltpu.async_copy(tab_hbm.at[src_v.at[ch1 + 1]], rows0_v, sem0)

            pltpu.make_async_copy(
                tab_hbm.at[src_v.at[ch1]], rows1_v, sem1).wait()
            pltpu.sync_copy(rows1_v, acc_sh.at[dst_v.at[ch1]], add=True)
            return carry

        lax.fori_loop(0, NC2 // 2, body, 0)
    plsc.subcore_barrier()
    pltpu.sync_copy(acc_sh.at[pl.ds(s * RP, RP)],
                    out_hbm.at[c, pl.ds(s * RP, RP)])


def _prep_body(degp_ref, x_ref, x1_ref, dis_ref):
    degp = degp_ref[...]
    deg = jnp.sum(degp.reshape(NC * NS, NP), axis=0) + 1.0
    dis = lax.rsqrt(deg)
    dis2 = jnp.broadcast_to(dis[:, None], (NP, D))
    dis_ref[...] = dis2
    x1_ref[0:N, :] = x_ref[...] * dis2[0:N, :]
    x1_ref[N:NP, :] = jnp.zeros((NP - N, D), jnp.float32)


_prep = pl.pallas_call(
    _prep_body,
    out_shape=(jax.ShapeDtypeStruct((NP, D), jnp.float32),
               jax.ShapeDtypeStruct((NP, D), jnp.float32)),
)

RB = 512
GB = NP // RB


def _mid_body(acc_ref, x1_ref, dis_ref, W1_ref, b1_ref, W2_ref, x2_ref):
    i = pl.program_id(0)
    dis = dis_ref[...]
    u1 = dis * (acc_ref[0] + acc_ref[1] + x1_ref[...])
    h1 = jnp.dot(u1, W1_ref[...], preferred_element_type=jnp.float32,
                 precision=lax.Precision.HIGHEST) + b1_ref[...]
    h1 = jnp.maximum(h1, 0.0)
    g2 = jnp.dot(dis[:, 0:1] * h1, W2_ref[...],
                 preferred_element_type=jnp.float32,
                 precision=lax.Precision.HIGHEST)
    rows = lax.broadcasted_iota(jnp.int32, (RB, D), 0) + i * RB
    x2_ref[...] = jnp.where(rows < N, g2, 0.0)


_mid = pl.pallas_call(
    _mid_body,
    grid=(GB,),
    in_specs=[
        pl.BlockSpec((NC, RB, D), lambda i: (0, i, 0)),
        pl.BlockSpec((RB, D), lambda i: (i, 0)),
        pl.BlockSpec((RB, D), lambda i: (i, 0)),
        pl.BlockSpec((D, DH), lambda i: (0, 0)),
        pl.BlockSpec((1, DH), lambda i: (0, 0)),
        pl.BlockSpec((DH, D), lambda i: (0, 0)),
    ],
    out_specs=pl.BlockSpec((RB, D), lambda i: (i, 0)),
    out_shape=jax.ShapeDtypeStruct((NP, D), jnp.float32),
)


def _fin_body(acc_ref, x2_ref, dis_ref, b2_ref, out_ref):
    u = dis_ref[...] * (acc_ref[0] + acc_ref[1] + x2_ref[...])
    out_ref[...] = jnp.maximum(u + b2_ref[...], 0.0)


_fin = pl.pallas_call(
    _fin_body,
    grid=(GB,),
    in_specs=[
        pl.BlockSpec((NC, RB, D), lambda i: (0, i, 0)),
        pl.BlockSpec((RB, D), lambda i: (i, 0)),
        pl.BlockSpec((RB, D), lambda i: (i, 0)),
        pl.BlockSpec((1, D), lambda i: (0, 0)),
    ],
    out_specs=pl.BlockSpec((RB, D), lambda i: (i, 0)),
    out_shape=jax.ShapeDtypeStruct((NP, D), jnp.float32),
)


def kernel(x, edge_index, W1, b1, W2, b2):
    pad = jnp.full((EP - E,), N, jnp.int32)
    src2 = jnp.concatenate([edge_index[0], pad]).reshape(EP // K, K)
    dst2 = jnp.concatenate([edge_index[1], pad]).reshape(EP // K, K)
    zerosN = jnp.zeros((NP,), jnp.float32)
    zerosD = jnp.zeros((RP, D), jnp.float32)

    degp = _deg_kernel(dst2, zerosN)
    x1, dis = _prep(degp, x)
    acc1 = _agg_kernel(x1, src2, dst2, zerosD)
    x2 = _mid(acc1, x1, dis, W1, b1.reshape(1, DH), W2)
    acc2 = _agg_kernel(x2, src2, dst2, zerosD)
    out = _fin(acc2, x2, dis, b2.reshape(1, D))
    return out[:N]


# speedup vs baseline: 11.1911x; 1.0757x over previous
import functools

import jax
import jax.numpy as jnp
from jax import lax
from jax.experimental import pallas as pl
from jax.experimental.pallas import tpu as pltpu
from jax.experimental.pallas import tpu_sc as plsc

N = 10000
D = 128
DH = 256
E = 320000
NP = 10240
K = 128
NC, NS = 2, 16
NW = NC * NS
EP = 327680
NCH = EP // (NW * K)
CH0, CH1 = 128, 32
PS = 32
NPASS0, NPASS1 = CH0 // PS, CH1 // PS
RP = NP // NS
DW = 16

_mesh = plsc.VectorSubcoreMesh(
    core_axis_name="c", subcore_axis_name="s", num_cores=NC, num_subcores=NS)


@functools.partial(
    pl.kernel,
    out_type=jax.ShapeDtypeStruct((NC, NS, NP), jnp.float32),
    mesh=_mesh,
    scratch_types=[
        pltpu.VMEM((NCH, K), jnp.int32),
        pltpu.VMEM((NP,), jnp.float32),
    ],
    compiler_params=pltpu.CompilerParams(needs_layout_passes=False),
)
def _deg_kernel(dst_hbm, zeros_hbm, out_hbm, idx_v, hist_v):
    c = lax.axis_index("c")
    s = lax.axis_index("s")
    w = c * NS + s
    pltpu.sync_copy(dst_hbm.at[pl.ds(w * NCH, NCH)], idx_v)
    pltpu.sync_copy(zeros_hbm, hist_v)
    ones_v = jnp.ones((16,), jnp.float32)

    def body(ch, carry):
        def inner(j, carry2):
            idx = idx_v[ch, pl.ds(j * 16, 16)]
            plsc.addupdate_scatter(hist_v, [idx], ones_v)
            return carry2

        return lax.fori_loop(0, K // 16, inner, carry)

    lax.fori_loop(0, NCH, body, 0)
    pltpu.sync_copy(hist_v, out_hbm.at[c, s])


@functools.partial(
    pl.kernel,
    out_type=jax.ShapeDtypeStruct((NC, NP, D), jnp.float32),
    mesh=_mesh,
    scratch_types=[
        pltpu.VMEM((PS, K), jnp.int32),
        pltpu.VMEM((PS, K), jnp.int32),
        pltpu.VMEM((K, D), jnp.float32),
        pltpu.VMEM((K, D), jnp.float32),
        pltpu.VMEM_SHARED((NP, D), jnp.float32),
        pltpu.SemaphoreType.DMA,
        pltpu.SemaphoreType.DMA,
    ],
)
def _agg_kernel(tab_hbm, src_hbm, dst_hbm, zeros_hbm, out_hbm,
                src_v, dst_v, rows0_v, rows1_v, acc_sh, sem0, sem1):
    c = lax.axis_index("c")
    s = lax.axis_index("s")
    base = jnp.where(c == 0, s * CH0, NS * CH0 + s * CH1)
    npass = jnp.where(c == 0, NPASS0, NPASS1)
    pltpu.sync_copy(zeros_hbm, acc_sh.at[pl.ds(s * RP, RP)])
    plsc.subcore_barrier()

    for p in range(max(NPASS0, NPASS1)):

        @pl.when(p < npass)
        def _pass():
            pltpu.sync_copy(src_hbm.at[pl.ds(base + p * PS, PS)], src_v)
            pltpu.sync_copy(dst_hbm.at[pl.ds(base + p * PS, PS)], dst_v)
            pltpu.async_copy(tab_hbm.at[src_v.at[0]], rows0_v, sem0)

            def body(i, carry):
                ch0 = 2 * i
                ch1 = 2 * i + 1
                pltpu.async_copy(tab_hbm.at[src_v.at[ch1]], rows1_v, sem1)
                pltpu.make_async_copy(
                    tab_hbm.at[src_v.at[ch0]], rows0_v, sem0).wait()
                pltpu.sync_copy(rows0_v, acc_sh.at[dst_v.at[ch0]], add=True)

                @pl.when(ch1 + 1 < PS)
                def _():
                    pltpu.async_copy(
                        tab_hbm.at[src_v.at[ch1 + 1]], rows0_v, sem0)

                pltpu.make_async_copy(
                    tab_hbm.at[src_v.at[ch1]], rows1_v, sem1).wait()
                pltpu.sync_copy(rows1_v, acc_sh.at[dst_v.at[ch1]], add=True)
                return carry

            lax.fori_loop(0, PS // 2, body, 0)

    plsc.subcore_barrier()
    pltpu.sync_copy(acc_sh.at[pl.ds(s * RP, RP)],
                    out_hbm.at[c, pl.ds(s * RP, RP)])


def _prep_body(degp_ref, x_ref, x1_ref, dis_ref):
    degp = degp_ref[...]
    deg = jnp.sum(degp.reshape(NC * NS, NP), axis=0) + 1.0
    dis = lax.rsqrt(deg)
    dis2 = jnp.broadcast_to(dis[:, None], (NP, D))
    dis_ref[...] = dis2
    x1_ref[0:N, :] = x_ref[...] * dis2[0:N, :]
    x1_ref[N:NP, :] = jnp.zeros((NP - N, D), jnp.float32)


_prep = pl.pallas_call(
    _prep_body,
    out_shape=(jax.ShapeDtypeStruct((NP, D), jnp.float32),
               jax.ShapeDtypeStruct((NP, D), jnp.float32)),
)

RB = 512
GB = NP // RB


def _mid_body(acc_ref, x1_ref, dis_ref, W1_ref, b1_ref, W2_ref, x2_ref):
    i = pl.program_id(0)
    dis = dis_ref[...]
    u1 = dis * (acc_ref[0] + acc_ref[1] + x1_ref[...])
    h1 = jnp.dot(u1, W1_ref[...], preferred_element_type=jnp.float32,
                 precision=lax.Precision.HIGHEST) + b1_ref[...]
    h1 = jnp.maximum(h1, 0.0)
    g2 = jnp.dot(dis[:, 0:1] * h1, W2_ref[...],
                 preferred_element_type=jnp.float32,
                 precision=lax.Precision.HIGHEST)
    rows = lax.broadcasted_iota(jnp.int32, (RB, D), 0) + i * RB
    x2_ref[...] = jnp.where(rows < N, g2, 0.0)


_mid = pl.pallas_call(
    _mid_body,
    grid=(GB,),
    in_specs=[
        pl.BlockSpec((NC, RB, D), lambda i: (0, i, 0)),
        pl.BlockSpec((RB, D), lambda i: (i, 0)),
        pl.BlockSpec((RB, D), lambda i: (i, 0)),
        pl.BlockSpec((D, DH), lambda i: (0, 0)),
        pl.BlockSpec((1, DH), lambda i: (0, 0)),
        pl.BlockSpec((DH, D), lambda i: (0, 0)),
    ],
    out_specs=pl.BlockSpec((RB, D), lambda i: (i, 0)),
    out_shape=jax.ShapeDtypeStruct((NP, D), jnp.float32),
)


def _fin_body(acc_ref, x2_ref, dis_ref, b2_ref, out_ref):
    u = dis_ref[...] * (acc_ref[0] + acc_ref[1] + x2_ref[...])
    out_ref[...] = jnp.maximum(u + b2_ref[...], 0.0)


_fin = pl.pallas_call(
    _fin_body,
    grid=(GB,),
    in_specs=[
        pl.BlockSpec((NC, RB, D), lambda i: (0, i, 0)),
        pl.BlockSpec((RB, D), lambda i: (i, 0)),
        pl.BlockSpec((RB, D), lambda i: (i, 0)),
        pl.BlockSpec((1, D), lambda i: (0, 0)),
    ],
    out_specs=pl.BlockSpec((RB, D), lambda i: (i, 0)),
    out_shape=jax.ShapeDtypeStruct((NP, D), jnp.float32),
)


def kernel(x, edge_index, W1, b1, W2, b2):
    pad = jnp.full((EP - E,), N, jnp.int32)
    src2 = jnp.concatenate([edge_index[0], pad]).reshape(EP // K, K)
    dst2 = jnp.concatenate([edge_index[1], pad]).reshape(EP // K, K)
    zerosN = jnp.zeros((NP,), jnp.float32)
    zerosD = jnp.zeros((RP, D), jnp.float32)

    degp = _deg_kernel(dst2, zerosN)
    x1, dis = _prep(degp, x)
    acc1 = _agg_kernel(x1, src2, dst2, zerosD)
    x2 = _mid(acc1, x1, dis, W1, b1.reshape(1, DH), W2)
    acc2 = _agg_kernel(x2, src2, dst2, zerosD)
    out = _fin(acc2, x2, dis, b2.reshape(1, D))
    return out[:N]
